# baseline (speedup 1.0000x reference)
import functools

import jax
import jax.numpy as jnp
from jax import lax
from jax.experimental import pallas as pl
from jax.experimental.pallas import tpu as pltpu
from jax.experimental.pallas import tpu_sc as plsc

B, L, D = 16, 4096, 300
GW = 1200
SENT_W = L * D
C = 16
CHW = C * GW
DPAD = 304
SENT_PER_CORE = 8
NSUB = 16

_mesh = plsc.VectorSubcoreMesh(core_axis_name="c", subcore_axis_name="s")


def _tree_sum(vals):
    while len(vals) > 1:
        nxt = [vals[i] + vals[i + 1] for i in range(0, len(vals) - 1, 2)]
        if len(vals) % 2:
            nxt.append(vals[-1])
        vals = nxt
    return vals[0]


@functools.partial(
    pl.kernel,
    mesh=_mesh,
    out_type=jax.ShapeDtypeStruct((B * DPAD,), jnp.float32),
    scratch_types=[
        pltpu.VMEM((CHW,), jnp.float32),
        pltpu.VMEM((GW,), jnp.float32),
        pltpu.VMEM((DPAD,), jnp.float32),
        pltpu.VMEM((DPAD,), jnp.float32),
        pltpu.VMEM((NSUB,), jnp.int32),
        pltpu.VMEM_SHARED((SENT_PER_CORE * NSUB, DPAD), jnp.float32),
    ],
    compiler_params=pltpu.CompilerParams(needs_layout_passes=False),
)
def _awe_pool(x_hbm, len_hbm, out_hbm, buf, acc, prow, tmp, lens_v, shared):
    sid = lax.axis_index("s")
    cid = lax.axis_index("c")
    zero16 = jnp.zeros((16,), jnp.float32)
    lane = lax.iota(jnp.int32, 16)

    pltpu.sync_copy(len_hbm, lens_v)

    def sent_len(gidx):
        return plsc.load_gather(lens_v, [jnp.full((16,), gidx, jnp.int32)])[0]

    def zero_acc():
        for j in range(GW // 16):
            acc[pl.ds(16 * j, 16)] = zero16

    def fold_acc_to_prow():
        for j in range(DPAD // 16):
            tot = zero16
            for p in range(4):
                ind = jnp.minimum(lane + (16 * j + D * p), GW - 1)
                tot = tot + plsc.load_gather(acc, [ind])
            prow[pl.ds(16 * j, 16)] = tot

    gcounts = []
    for s in range(SENT_PER_CORE):
        ln = sent_len(cid * SENT_PER_CORE + s)
        gcounts.append((ln + 3) // 4)
    total = _tree_sum(list(gcounts))
    quota = (total + NSUB - 1) // NSUB
    lo = jnp.minimum(sid * quota, total)
    hi = jnp.minimum(lo + quota, total)

    base = jnp.int32(0)
    for s in range(SENT_PER_CORE):
        gs = cid * SENT_PER_CORE + s
        gcount = gcounts[s]
        a = jnp.clip(lo - base, 0, gcount)
        b = jnp.clip(hi - base, 0, gcount)
        lenw = sent_len(gs) * D
        sent_off = gs * SENT_W
        zero_acc()

        @pl.when(b > a)
        def _process(a=a, b=b, lenw=lenw, sent_off=sent_off, gcount=gcount):
            ragged = jnp.logical_and(b == gcount, lenw < gcount * GW)
            nfull = (b - a - ragged.astype(jnp.int32)) // C
            ntail = (b - a) - nfull * C

            def chunk_body(i, carry):
                g = a + i * C
                pltpu.sync_copy(x_hbm.at[pl.ds(sent_off + g * GW, CHW)], buf)

                def jbody(j, jc):
                    o = j * 16
                    vals = [buf[pl.ds(o + gg * GW, 16)] for gg in range(C)]
                    acc[pl.ds(o, 16)] = acc[pl.ds(o, 16)] + _tree_sum(vals)
                    return jc

                lax.fori_loop(0, GW // 16, jbody, 0)
                return carry

            lax.fori_loop(0, nfull, chunk_body, 0)

            def tail_body(i, carry):
                g = a + nfull * C + i
                pltpu.sync_copy(
                    x_hbm.at[pl.ds(sent_off + g * GW, GW)], buf.at[pl.ds(0, GW)]
                )
                bnd = jnp.minimum(lenw - g * GW, GW)
                start = (bnd // 16) * 16
                nz = (GW - start) // 16

                def zbody(k, kc):
                    off = start + k * 16
                    keep = (off + lane) < bnd
                    buf[pl.ds(off, 16)] = jnp.where(keep, buf[pl.ds(off, 16)], 0.0)
                    return kc

                lax.fori_loop(0, nz, zbody, 0)

                def jbody(j, jc):
                    o = j * 16
                    acc[pl.ds(o, 16)] = acc[pl.ds(o, 16)] + buf[pl.ds(o, 16)]
                    return jc

                lax.fori_loop(0, GW // 16, jbody, 0)
                return carry

            lax.fori_loop(0, ntail, tail_body, 0)

        fold_acc_to_prow()
        pltpu.sync_copy(prow, shared.at[s * NSUB + sid])
        base = base + gcount

    plsc.subcore_barrier()

    @pl.when(sid < SENT_PER_CORE)
    def _fold():
        gs = cid * SENT_PER_CORE + sid
        lenf = plsc.load_gather(
            lens_v, [jnp.full((16,), gs, jnp.int32)]
        ).astype(jnp.float32)
        for j in range(DPAD // 16):
            prow[pl.ds(16 * j, 16)] = zero16
        for w in range(NSUB):
            pltpu.sync_copy(shared.at[sid * NSUB + w], tmp)
            for j in range(DPAD // 16):
                o = j * 16
                prow[pl.ds(o, 16)] = prow[pl.ds(o, 16)] + tmp[pl.ds(o, 16)]
        for j in range(DPAD // 16):
            o = j * 16
            prow[pl.ds(o, 16)] = prow[pl.ds(o, 16)] / lenf
        pltpu.sync_copy(prow, out_hbm.at[pl.ds(gs * DPAD, DPAD)])


def kernel(sentences, sentence_lengths):
    x = sentences.reshape(-1)
    out = _awe_pool(x, sentence_lengths)
    return out.reshape(B, DPAD)[:, :D]


# speedup vs baseline: 1.8357x; 1.8357x over previous
import functools

import jax
import jax.numpy as jnp
from jax import lax
from jax.experimental import pallas as pl
from jax.experimental.pallas import tpu as pltpu
from jax.experimental.pallas import tpu_sc as plsc

B, L, D = 16, 4096, 300
BLK = 8
CT = 128
CB = CT // BLK
DPAD = 304
NJ = 18
SENT_PER_CORE = 8
NSUB = 16

_mesh = plsc.VectorSubcoreMesh(core_axis_name="c", subcore_axis_name="s")


def _tree_sum(vals):
    while len(vals) > 1:
        nxt = [vals[i] + vals[i + 1] for i in range(0, len(vals) - 1, 2)]
        if len(vals) % 2:
            nxt.append(vals[-1])
        vals = nxt
    return vals[0]


@functools.partial(
    pl.kernel,
    mesh=_mesh,
    out_type=jax.ShapeDtypeStruct((2 * NSUB, SENT_PER_CORE, DPAD), jnp.float32),
    scratch_types=[
        pltpu.VMEM((CT, D), jnp.float32),
        pltpu.VMEM((CT, D), jnp.float32),
        pltpu.VMEM((SENT_PER_CORE, DPAD), jnp.float32),
        pltpu.VMEM((NSUB,), jnp.int32),
        pltpu.SemaphoreType.DMA,
        pltpu.SemaphoreType.DMA,
    ],
    compiler_params=pltpu.CompilerParams(needs_layout_passes=False),
)
def _awe_pool(x_hbm, len_hbm, part_hbm, bufa, bufb, prows, lens_v, sema, semb):
    sid = lax.axis_index("s")
    cid = lax.axis_index("c")
    zero16 = jnp.zeros((16,), jnp.float32)
    lane = lax.iota(jnp.int32, 16)

    pltpu.sync_copy(len_hbm, lens_v)

    def sent_len(gidx):
        return plsc.load_gather(lens_v, [jnp.full((16,), gidx, jnp.int32)])[0]

    def add_token(buf, t, accs):
        out = [accs[j] + buf[t, pl.ds(16 * j, 16)] for j in range(NJ)]
        v = buf[t, pl.ds(D - 16, 16)]
        out.append(accs[NJ] + jnp.where(lane >= 4, v, 0.0))
        return tuple(out)

    def acc_chunk(buf, accs):
        def q4(q, accs):
            t = q * 4
            for u in range(4):
                accs = add_token(buf, t + u, accs)
            return accs

        return lax.fori_loop(0, CT // 4, q4, accs)

    gcounts = []
    for s in range(SENT_PER_CORE):
        ln = sent_len(cid * SENT_PER_CORE + s)
        gcounts.append((ln + BLK - 1) // BLK)
    total = _tree_sum(list(gcounts))
    quota = (total + NSUB - 1) // NSUB
    lo = jnp.minimum(sid * quota, total)
    hi = jnp.minimum(lo + quota, total)

    base = jnp.int32(0)
    for s in range(SENT_PER_CORE):
        gs = cid * SENT_PER_CORE + s
        gcount = gcounts[s]
        a = jnp.clip(lo - base, 0, gcount)
        b = jnp.clip(hi - base, 0, gcount)
        lenT = sent_len(gs)

        def start(buf, sem, i, a=a, gs=gs):
            t0 = (a + i * CB) * BLK
            pltpu.async_copy(x_hbm.at[gs, pl.ds(t0, CT), :], buf, sem)

        def wait(buf, sem, gs=gs):
            pltpu.make_async_copy(x_hbm.at[gs, pl.ds(0, CT), :], buf, sem).wait()

        def compute(a=a, b=b, lenT=lenT, gs=gs, start=start, wait=wait):
            accs = tuple(zero16 for _ in range(NJ + 1))
            ragged = (b * BLK > lenT).astype(jnp.int32)
            nfull = (b - a - ragged) // CB
            ntail = (b - a) - nfull * CB

            @pl.when(nfull > 0)
            def _():
                start(bufa, sema, 0)

            @pl.when(nfull > 1)
            def _():
                start(bufb, semb, 1)

            def pair_body(p, accs):
                i0 = 2 * p
                wait(bufa, sema)

                @pl.when(i0 + 2 < nfull)
                def _():
                    start(bufa, sema, i0 + 2)

                accs = acc_chunk(bufa, accs)
                wait(bufb, semb)

                @pl.when(i0 + 3 < nfull)
                def _():
                    start(bufb, semb, i0 + 3)

                return acc_chunk(bufb, accs)

            accs = lax.fori_loop(0, nfull // 2, pair_body, accs)

            def odd_chunk(accs):
                wait(bufa, sema)
                return acc_chunk(bufa, accs)

            accs = lax.cond(nfull % 2 == 1, odd_chunk, lambda x: x, accs)

            def tail_body(i, accs):
                blk = a + nfull * CB + i
                t0 = blk * BLK
                pltpu.sync_copy(
                    x_hbm.at[gs, pl.ds(t0, BLK), :], bufa.at[pl.ds(0, BLK), :]
                )
                nv = jnp.clip(lenT - t0, 0, BLK)

                def tb(t, accs):
                    return add_token(bufa, t, accs)

                return lax.fori_loop(0, nv, tb, accs)

            return lax.fori_loop(0, ntail, tail_body, accs)

        def empty():
            return tuple(zero16 for _ in range(NJ + 1))

        accs = lax.cond(b > a, compute, empty)

        for j in range(NJ + 1):
            prows[s, pl.ds(16 * j, 16)] = accs[j]
        base = base + gcount

    pltpu.sync_copy(prows, part_hbm.at[cid * NSUB + sid])


def _combine_body(part_ref, len_ref, out_ref):
    for c in range(2):
        acc = part_ref[c * NSUB]
        for w in range(1, NSUB):
            acc = acc + part_ref[c * NSUB + w]
        head = acc[:, : NJ * 16]
        tail = acc[:, NJ * 16 + 4:]
        pad = jnp.zeros((SENT_PER_CORE, 4), jnp.float32)
        full = jnp.concatenate([head, tail, pad], axis=1)
        out_ref[pl.ds(c * SENT_PER_CORE, SENT_PER_CORE), :] = (
            full / len_ref[pl.ds(c * SENT_PER_CORE, SENT_PER_CORE), :]
        )


_combine = pl.pallas_call(
    _combine_body,
    out_shape=jax.ShapeDtypeStruct((B, DPAD), jnp.float32),
)


def kernel(sentences, sentence_lengths):
    parts = _awe_pool(sentences, sentence_lengths)
    lens_f = sentence_lengths.astype(jnp.float32).reshape(B, 1)
    out = _combine(parts, lens_f)
    return out[:, :D]


# speedup vs baseline: 1.9666x; 1.0713x over previous
import functools

import jax
import jax.numpy as jnp
from jax import lax
from jax.experimental import pallas as pl
from jax.experimental.pallas import tpu as pltpu
from jax.experimental.pallas import tpu_sc as plsc

B, L, D = 16, 4096, 300
BLK = 8
CT = 128
CB = CT // BLK
DPAD = 304
NJ = 18
NSUB = 16
NW = 2 * NSUB

_mesh = plsc.VectorSubcoreMesh(core_axis_name="c", subcore_axis_name="s")


@functools.partial(
    pl.kernel,
    mesh=_mesh,
    out_type=jax.ShapeDtypeStruct((NW, B, DPAD), jnp.float32),
    scratch_types=[
        pltpu.VMEM((CT, D), jnp.float32),
        pltpu.VMEM((CT, D), jnp.float32),
        pltpu.VMEM((B, DPAD), jnp.float32),
        pltpu.VMEM((B,), jnp.int32),
        pltpu.VMEM((B,), jnp.int32),
        pltpu.SemaphoreType.DMA,
        pltpu.SemaphoreType.DMA,
    ],
    compiler_params=pltpu.CompilerParams(needs_layout_passes=False),
)
def _awe_pool(x_hbm, len_hbm, part_hbm, bufa, bufb, prows, lens_v, gcnt_v,
              sema, semb):
    sid = lax.axis_index("s")
    cid = lax.axis_index("c")
    wid = cid * NSUB + sid
    zero16 = jnp.zeros((16,), jnp.float32)
    lane = lax.iota(jnp.int32, 16)

    pltpu.sync_copy(len_hbm, lens_v)
    lens_vec = lens_v[...]
    gvec = (lens_vec + (BLK - 1)) // BLK
    gcnt_v[...] = gvec

    def vext(ref, gidx):
        return plsc.load_gather(ref, [jnp.full((16,), gidx, jnp.int32)])[0]

    def add_token(buf, t, accs):
        out = [accs[j] + buf[t, pl.ds(16 * j, 16)] for j in range(NJ)]
        v = buf[t, pl.ds(D - 16, 16)]
        out.append(accs[NJ] + jnp.where(lane >= 4, v, 0.0))
        return tuple(out)

    def acc_chunk(buf, accs):
        def q4(q, accs):
            t = q * 4
            for u in range(4):
                accs = add_token(buf, t + u, accs)
            return accs

        return lax.fori_loop(0, CT // 4, q4, accs)

    total = jnp.int32(0)
    for s in range(B):
        total = total + vext(gcnt_v, s)
    quota = (total + NW - 1) // NW
    lo = jnp.minimum(wid * quota, total)
    hi = jnp.minimum(lo + quota, total)

    def sent_body(s, base):
        gcount = vext(gcnt_v, s)
        a = jnp.clip(lo - base, 0, gcount)
        b = jnp.clip(hi - base, 0, gcount)
        lenT = vext(lens_v, s)

        def start(buf, sem, i, a=a, s=s):
            t0 = (a + i * CB) * BLK
            pltpu.async_copy(x_hbm.at[s, pl.ds(t0, CT), :], buf, sem)

        def wait(buf, sem, s=s):
            pltpu.make_async_copy(x_hbm.at[s, pl.ds(0, CT), :], buf, sem).wait()

        def compute(a=a, b=b, lenT=lenT, s=s, start=start, wait=wait):
            accs = tuple(zero16 for _ in range(NJ + 1))
            ragged = (b * BLK > lenT).astype(jnp.int32)
            nfull = (b - a - ragged) // CB
            ntail = (b - a) - nfull * CB

            @pl.when(nfull > 0)
            def _():
                start(bufa, sema, 0)

            @pl.when(nfull > 1)
            def _():
                start(bufb, semb, 1)

            def pair_body(p, accs):
                i0 = 2 * p
                wait(bufa, sema)

                @pl.when(i0 + 2 < nfull)
                def _():
                    start(bufa, sema, i0 + 2)

                accs = acc_chunk(bufa, accs)
                wait(bufb, semb)

                @pl.when(i0 + 3 < nfull)
                def _():
                    start(bufb, semb, i0 + 3)

                return acc_chunk(bufb, accs)

            accs = lax.fori_loop(0, nfull // 2, pair_body, accs)

            def odd_chunk(accs):
                wait(bufa, sema)
                return acc_chunk(bufa, accs)

            accs = lax.cond(nfull % 2 == 1, odd_chunk, lambda x: x, accs)

            def tail_body(i, accs):
                blk = a + nfull * CB + i
                t0 = blk * BLK
                pltpu.sync_copy(
                    x_hbm.at[s, pl.ds(t0, BLK), :], bufa.at[pl.ds(0, BLK), :]
                )
                nv = jnp.clip(lenT - t0, 0, BLK)

                def tb(t, accs):
                    return add_token(bufa, t, accs)

                return lax.fori_loop(0, nv, tb, accs)

            return lax.fori_loop(0, ntail, tail_body, accs)

        def empty():
            return tuple(zero16 for _ in range(NJ + 1))

        accs = lax.cond(b > a, compute, empty)

        for j in range(NJ + 1):
            prows[s, pl.ds(16 * j, 16)] = accs[j]
        return base + gcount

    lax.fori_loop(0, B, sent_body, jnp.int32(0))

    pltpu.sync_copy(prows, part_hbm.at[wid])


def _combine_body(part_ref, len_ref, out_ref):
    acc = part_ref[0]
    for w in range(1, NW):
        acc = acc + part_ref[w]
    head = acc[:, : NJ * 16]
    tail = acc[:, NJ * 16 + 4:]
    pad = jnp.zeros((B, 4), jnp.float32)
    full = jnp.concatenate([head, tail, pad], axis=1)
    out_ref[...] = full / len_ref[...]


_combine = pl.pallas_call(
    _combine_body,
    out_shape=jax.ShapeDtypeStruct((B, DPAD), jnp.float32),
)


def kernel(sentences, sentence_lengths):
    parts = _awe_pool(sentences, sentence_lengths)
    lens_f = sentence_lengths.astype(jnp.float32).reshape(B, 1)
    out = _combine(parts, lens_f)
    return out[:, :D]
